# baseline (speedup 1.0000x reference)
import functools

import jax
import jax.numpy as jnp
from jax import lax
from jax.experimental import pallas as pl
from jax.experimental.pallas import tpu as pltpu
from jax.experimental.pallas import tpu_sc as plsc

V = 100000
D = 128
B = 4096
MOM = 0.5
NC, NS, L = 2, 16, 16
NW = NC * NS
RSMALL = 3120
RTAIL = 8
NBIG = 20
CH = 128
CAP = B + 2 * L
TRASH = CAP - 1
DB = D // L

_mesh = plsc.VectorSubcoreMesh(core_axis_name="c", subcore_axis_name="s")


@functools.partial(
    pl.kernel,
    out_type=jax.ShapeDtypeStruct((V, D), jnp.float32),
    mesh=_mesh,
    compiler_params=pltpu.CompilerParams(needs_layout_passes=False),
    scratch_types=[
        pltpu.VMEM((B,), jnp.int32),
        pltpu.VMEM((CAP,), jnp.int32),
        pltpu.VMEM((CAP,), jnp.int32),
        pltpu.VMEM((CAP,), jnp.int32),
        pltpu.VMEM((B // CH, CH), jnp.int32),
        pltpu.VMEM((CH, D), jnp.float32),
        pltpu.VMEM((CH, D), jnp.float32),
        pltpu.SemaphoreType.DMA,
        pltpu.SemaphoreType.DMA,
        pltpu.SemaphoreType.DMA,
        pltpu.SemaphoreType.DMA,
    ],
)
def _sc_update(x_hbm, y_hbm, mem_hbm, out_hbm,
               y_v, pos_v, idx_v, last_v, idx2, mrow, xrow,
               semA, semB, semC, semD):
    wid = lax.axis_index("s") * NC + lax.axis_index("c")
    big = wid < NBIG
    lo = pl.multiple_of(wid * RSMALL + jnp.minimum(wid, NBIG) * RTAIL, 8)
    lo2 = pl.multiple_of(lo + RSMALL, 8)
    hi = lo + RSMALL + jnp.where(big, RTAIL, 0)
    lanes = lax.iota(jnp.int32, L)

    slab = pltpu.make_async_copy(mem_hbm.at[pl.ds(lo, RSMALL)],
                                 out_hbm.at[pl.ds(lo, RSMALL)], semC)
    slab_b = pltpu.make_async_copy(mem_hbm.at[pl.ds(lo2, RTAIL)],
                                   out_hbm.at[pl.ds(lo2, RTAIL)], semD)
    slab.start()

    @pl.when(big)
    def _():
        slab_b.start()

    pltpu.sync_copy(y_hbm, y_v)

    @pl.loop(0, B // L, init_carry=jnp.int32(0))
    def compact(i, cnt):
        yv = y_v[pl.ds(i * L, L)]
        m = (yv >= jnp.full((L,), lo, jnp.int32)) & (yv < jnp.full((L,), hi, jnp.int32))
        mi = jnp.where(m, jnp.full((L,), 1, jnp.int32), jnp.full((L,), 0, jnp.int32))
        slots = jnp.where(m, plsc.cumsum(mi) + jnp.full((L,), cnt - 1, jnp.int32),
                          jnp.full((L,), TRASH, jnp.int32))
        plsc.store_scatter(pos_v, [slots], i * L + lanes)
        plsc.store_scatter(idx_v, [slots], yv)
        return cnt + jnp.sum(mi)

    cnt = compact

    @pl.when(cnt > 0)
    def _():
        nch = (cnt + CH - 1) // CH
        pend = nch * CH

        last_idx = idx_v[pl.ds(cnt - 1, L)][0]
        last_pos = pos_v[pl.ds(cnt - 1, L)][0]

        @pl.loop(0, pend // L)
        def fill(b):
            base = b * L
            slot = base + lanes
            live = slot < jnp.full((L,), cnt, jnp.int32)
            cur_i = idx_v[pl.ds(base, L)]
            cur_p = pos_v[pl.ds(base, L)]
            idx_v[pl.ds(base, L)] = jnp.where(live, cur_i, jnp.full((L,), last_idx, jnp.int32))
            pos_v[pl.ds(base, L)] = jnp.where(live, cur_p, jnp.full((L,), last_pos, jnp.int32))
            last_v[pl.ds(base, L)] = slot

        @pl.loop(0, pend)
        def dedup(k):
            vk = idx_v[pl.ds(k, L)][0]

            @pl.loop(0, pend // L)
            def blk(b):
                base = b * L
                eq = idx_v[pl.ds(base, L)] == jnp.full((L,), vk, jnp.int32)
                cur = last_v[pl.ds(base, L)]
                last_v[pl.ds(base, L)] = jnp.where(eq, jnp.full((L,), k, jnp.int32), cur)

        @pl.loop(0, pend // L)
        def rewrite(b):
            base = b * L
            w = last_v[pl.ds(base, L)]
            pos_v[pl.ds(base, L)] = plsc.load_gather(pos_v, [w])

        @pl.loop(0, nch)
        def chunk(c):
            off = c * CH
            for b in range(CH // L):
                idx2[c, pl.ds(b * L, L)] = idx_v[pl.ds(off + b * L, L)]
            gm = pltpu.async_copy(mem_hbm.at[idx2.at[c]], mrow, semA)
            gx = pltpu.async_copy(x_hbm.at[pos_v.at[pl.ds(off, CH)]], xrow, semB)
            gm.wait()
            gx.wait()

            @pl.loop(0, CH)
            def row(r):
                acc = jnp.zeros((L,), jnp.float32)
                for dblk in range(DB):
                    s = pl.ds(dblk * L, L)
                    v = mrow[r, s] * MOM + xrow[r, s] * (1.0 - MOM)
                    mrow[r, s] = v
                    acc = acc + v * v
                ss = jnp.full((L,), jnp.sum(acc), jnp.float32)
                bits = plsc.bitcast(ss, jnp.int32)
                guess = plsc.bitcast(
                    jnp.full((L,), 0x5F3759DF, jnp.int32) - (bits >> 1),
                    jnp.float32)
                for _ in range(3):
                    guess = guess * (1.5 - 0.5 * ss * guess * guess)
                for dblk in range(DB):
                    s = pl.ds(dblk * L, L)
                    mrow[r, s] = mrow[r, s] * guess

            @pl.when(c == 0)
            def _():
                slab.wait()

                @pl.when(big)
                def _():
                    slab_b.wait()

            sc = pltpu.async_copy(mrow, out_hbm.at[idx2.at[c]], semA)
            sc.wait()

    @pl.when(cnt == 0)
    def _():
        slab.wait()

        @pl.when(big)
        def _():
            slab_b.wait()


def kernel(x, x2, y, memory):
    new_memory = _sc_update(x, y, memory)
    return (x, x2, new_memory)


# speedup vs baseline: 15.8218x; 15.8218x over previous
import functools

import jax
import jax.numpy as jnp
from jax import lax
from jax.experimental import pallas as pl
from jax.experimental.pallas import tpu as pltpu
from jax.experimental.pallas import tpu_sc as plsc

V = 100000
D = 128
B = 4096
MOM = 0.5
NC, NS, L = 2, 16, 16
NW = NC * NS
R = V // NW
RCAP = 3136
TRASHR = RCAP - 1
CH = 128
CAP = B + 2 * L
TRASH = CAP - 1
DB = D // L

_mesh = plsc.VectorSubcoreMesh(core_axis_name="c", subcore_axis_name="s")


@functools.partial(
    pl.kernel,
    out_type=(),
    mesh=_mesh,
    compiler_params=pltpu.CompilerParams(needs_layout_passes=False),
    scratch_types=[
        pltpu.VMEM((B,), jnp.int32),
        pltpu.VMEM((RCAP,), jnp.int32),
        pltpu.VMEM((CAP,), jnp.int32),
        pltpu.VMEM((CAP,), jnp.int32),
        pltpu.VMEM((B // CH, CH), jnp.int32),
        pltpu.VMEM((CH, D), jnp.float32),
        pltpu.VMEM((CH, D), jnp.float32),
        pltpu.SemaphoreType.DMA,
        pltpu.SemaphoreType.DMA,
    ],
)
def _sc_update(x_hbm, y_hbm, mem_hbm, out_ref,
               y_v, win_v, pos_v, idx_v, idx2, mrow, xrow, semA, semB):
    wid = lax.axis_index("s") * NC + lax.axis_index("c")
    lo = wid * R
    hi = lo + R
    lanes = lax.iota(jnp.int32, L)
    onev = jnp.full((L,), 1, jnp.int32)
    zerov = jnp.full((L,), 0, jnp.int32)
    lov = jnp.full((L,), lo, jnp.int32)
    hiv = jnp.full((L,), hi, jnp.int32)
    sinkr = jnp.full((L,), TRASHR, jnp.int32)

    pltpu.sync_copy(y_hbm, y_v)

    @pl.loop(0, RCAP // L)
    def clear(b):
        win_v[pl.ds(b * L, L)] = jnp.full((L,), -1, jnp.int32)

    @pl.loop(0, B // L)
    def winners(i):
        yv = y_v[pl.ds(i * L, L)]
        m = (yv >= lov) & (yv < hiv)
        local = jnp.where(m, yv - lov, sinkr)
        pos = i * L + lanes
        plsc.store_scatter(win_v, [local], pos)
        g = plsc.load_gather(win_v, [local])
        bad0 = m & (g < pos)
        nb0 = jnp.sum(jnp.where(bad0, onev, zerov))

        def cond(carry):
            return carry[0] > 0

        def body(carry):
            _, bad = carry
            slots = jnp.where(bad, local, sinkr)
            plsc.store_scatter(win_v, [slots], pos)
            g2 = plsc.load_gather(win_v, [slots])
            bad2 = bad & (g2 < pos)
            return (jnp.sum(jnp.where(bad2, onev, zerov)), bad2)

        lax.while_loop(cond, body, (nb0, bad0))

    @pl.loop(0, B // L, init_carry=jnp.int32(0))
    def compact(i, cnt):
        yv = y_v[pl.ds(i * L, L)]
        m = (yv >= lov) & (yv < hiv)
        local = jnp.where(m, yv - lov, sinkr)
        pos = i * L + lanes
        g = plsc.load_gather(win_v, [local])
        win = m & (g == pos)
        mi = jnp.where(win, onev, zerov)
        slots = jnp.where(win, plsc.cumsum(mi) + jnp.full((L,), cnt - 1, jnp.int32),
                          jnp.full((L,), TRASH, jnp.int32))
        plsc.store_scatter(pos_v, [slots], pos)
        plsc.store_scatter(idx_v, [slots], yv)
        return cnt + jnp.sum(mi)

    cnt = compact

    @pl.when(cnt > 0)
    def _():
        nch = (cnt + CH - 1) // CH
        pend = nch * CH

        last_idx = idx_v[pl.ds(cnt - 1, L)][0]
        last_pos = pos_v[pl.ds(cnt - 1, L)][0]

        @pl.loop(cnt // L, pend // L)
        def fill(b):
            base = b * L
            live = base + lanes < jnp.full((L,), cnt, jnp.int32)
            cur_i = idx_v[pl.ds(base, L)]
            cur_p = pos_v[pl.ds(base, L)]
            idx_v[pl.ds(base, L)] = jnp.where(live, cur_i, jnp.full((L,), last_idx, jnp.int32))
            pos_v[pl.ds(base, L)] = jnp.where(live, cur_p, jnp.full((L,), last_pos, jnp.int32))

        @pl.loop(0, nch)
        def chunk(c):
            off = c * CH
            for b in range(CH // L):
                idx2[c, pl.ds(b * L, L)] = idx_v[pl.ds(off + b * L, L)]
            gm = pltpu.async_copy(mem_hbm.at[idx2.at[c]], mrow, semA)
            gx = pltpu.async_copy(x_hbm.at[pos_v.at[pl.ds(off, CH)]], xrow, semB)
            gm.wait()
            gx.wait()

            @pl.loop(0, CH)
            def row(r):
                acc = jnp.zeros((L,), jnp.float32)
                vs = []
                for dblk in range(DB):
                    s = pl.ds(dblk * L, L)
                    v = mrow[r, s] * MOM + xrow[r, s] * (1.0 - MOM)
                    vs.append(v)
                    acc = acc + v * v
                ss = jnp.full((L,), jnp.sum(acc), jnp.float32)
                bits = plsc.bitcast(ss, jnp.int32)
                guess = plsc.bitcast(
                    jnp.full((L,), 0x5F3759DF, jnp.int32) - (bits >> 1),
                    jnp.float32)
                for _ in range(3):
                    guess = guess * (1.5 - 0.5 * ss * guess * guess)
                for dblk in range(DB):
                    mrow[r, pl.ds(dblk * L, L)] = vs[dblk] * guess

            sc = pltpu.async_copy(mrow, out_ref.at[idx2.at[c]], semA)
            sc.wait()


def kernel(x, x2, y, memory):
    mem_ref = jax.new_ref(memory)
    _sc_update(x, y, memory, mem_ref)
    return (x, x2, mem_ref[...])
